# baseline (speedup 1.0000x reference)
import jax
import jax.numpy as jnp
from jax import lax
from jax.experimental import pallas as pl

N_CLASSES = 1000


def _onehot_body(x_ref, o_ref):
    idx = x_ref[...]
    classes = lax.broadcasted_iota(jnp.int32, o_ref.shape, 1)
    o_ref[...] = (classes == idx).astype(jnp.float32)


def kernel(x):
    B, S = x.shape
    N = B * S
    xf = x.reshape(N, 1).astype(jnp.int32)
    R = 512
    out = pl.pallas_call(
        _onehot_body,
        grid=(N // R,),
        in_specs=[pl.BlockSpec((R, 1), lambda i: (i, 0))],
        out_specs=pl.BlockSpec((R, N_CLASSES), lambda i: (i, 0)),
        out_shape=jax.ShapeDtypeStruct((N, N_CLASSES), jnp.float32),
    )(xf)
    return out.reshape(B, S, N_CLASSES)


# speedup vs baseline: 1.4000x; 1.4000x over previous
import jax
import jax.numpy as jnp
from jax import lax
from jax.experimental import pallas as pl

N_CLASSES = 1000


def _onehot_body(x_ref, o_ref):
    idx = x_ref[...]
    classes = lax.broadcasted_iota(jnp.int32, o_ref.shape, 2)
    o_ref[...] = (classes == idx).astype(jnp.float32)


def kernel(x):
    B, S = x.shape
    x3 = x.reshape(B, S, 1).astype(jnp.int32)
    Rb = 32
    out = pl.pallas_call(
        _onehot_body,
        grid=(B // Rb,),
        in_specs=[pl.BlockSpec((Rb, S, 1), lambda i: (i, 0, 0))],
        out_specs=pl.BlockSpec((Rb, S, N_CLASSES), lambda i: (i, 0, 0)),
        out_shape=jax.ShapeDtypeStruct((B, S, N_CLASSES), jnp.float32),
    )(x3)
    return out


# speedup vs baseline: 1.4337x; 1.0241x over previous
import jax
import jax.numpy as jnp
from jax import lax
from jax.experimental import pallas as pl

N_CLASSES = 1000


def _onehot_body(x_ref, o_ref):
    idx = x_ref[...]
    classes = lax.broadcasted_iota(jnp.int32, o_ref.shape, 2)
    o_ref[...] = (classes == idx).astype(jnp.float32)


def kernel(x):
    B, S = x.shape
    x3 = x.reshape(B, S, 1).astype(jnp.int32)
    Rb = 128
    out = pl.pallas_call(
        _onehot_body,
        grid=(B // Rb,),
        in_specs=[pl.BlockSpec((Rb, S, 1), lambda i: (i, 0, 0))],
        out_specs=pl.BlockSpec((Rb, S, N_CLASSES), lambda i: (i, 0, 0)),
        out_shape=jax.ShapeDtypeStruct((B, S, N_CLASSES), jnp.float32),
    )(x3)
    return out
